# Initial kernel scaffold; baseline (speedup 1.0000x reference)
import jax
import jax.numpy as jnp
from jax.experimental import pallas as pl


def kernel(ori, embeds):
    raise NotImplementedError("write your pallas kernel here")


# speedup vs baseline: 15.2606x; 15.2606x over previous
import functools
import jax
import jax.numpy as jnp
from jax import lax
from jax.experimental import pallas as pl
from jax.experimental.pallas import tpu as pltpu
from jax.experimental.pallas import tpu_sc as plsc

_NUM_EMBED = 8192
_D = 1024
_N = 65536
_NW = 32
_PPW = _N // _NW
_CHUNK = 32
_NCHUNK = _PPW // _CHUNK
_GROUPS = _PPW // 16


def _body(ori_hbm, emb_hbm, out_hbm,
          ori_v, idx_l, idx_r, w_a, w_b, rows_l, rows_r, out_v, sem):
    wid = lax.axis_index("s") * 2 + lax.axis_index("c")
    base = wid * _PPW

    pltpu.sync_copy(ori_hbm.at[pl.ds(base, _PPW)], ori_v)

    def idx_body(j, _):
        ov = ori_v[pl.ds(j * 16, 16)]
        o = (ov + 1.0) * (0.5 * _NUM_EMBED)
        f = o.astype(jnp.int32)
        ff = f.astype(jnp.float32)
        cf = jnp.where(o > ff, ff + 1.0, ff)
        c = cf.astype(jnp.int32)
        idx_l[pl.ds(j * 16, 16)] = c - 1
        idx_r[pl.ds(j * 16, 16)] = c & (_NUM_EMBED - 1)
        w_a[pl.ds(j * 16, 16)] = cf - o
        w_b[pl.ds(j * 16, 16)] = o - (cf - 1.0)
        return 0

    lax.fori_loop(0, _GROUPS, idx_body, 0, unroll=4)

    def chunk_body(g, _):
        cl = pltpu.async_copy(emb_hbm.at[idx_l.at[pl.ds(g * _CHUNK, _CHUNK)]],
                              rows_l, sem)
        cr = pltpu.async_copy(emb_hbm.at[idx_r.at[pl.ds(g * _CHUNK, _CHUNK)]],
                              rows_r, sem)
        cl.wait()
        cr.wait()

        def point_body(p, _):
            a = jnp.full((16,), w_a[pl.ds(g * _CHUNK + p, 16)][0],
                         dtype=jnp.float32)
            b = jnp.full((16,), w_b[pl.ds(g * _CHUNK + p, 16)][0],
                         dtype=jnp.float32)

            def col_body(jj, _):
                lv = rows_l[p, pl.ds(jj * 16, 16)]
                rv = rows_r[p, pl.ds(jj * 16, 16)]
                out_v[p, pl.ds(jj * 16, 16)] = lv * a + rv * b
                return 0

            lax.fori_loop(0, _D // 16, col_body, 0, unroll=4)
            return 0

        lax.fori_loop(0, _CHUNK, point_body, 0)
        pltpu.sync_copy(out_v, out_hbm.at[pl.ds(base + g * _CHUNK, _CHUNK)])
        return 0

    lax.fori_loop(0, _NCHUNK, chunk_body, 0)


@jax.jit
def kernel(ori, embeds):
    emb2d = embeds.reshape(_NUM_EMBED, _D)
    mesh = plsc.VectorSubcoreMesh(core_axis_name="c", subcore_axis_name="s")
    out = pl.kernel(
        _body,
        out_type=jax.ShapeDtypeStruct((_N, _D), jnp.float32),
        mesh=mesh,
        scratch_types=[
            pltpu.VMEM((_PPW,), jnp.float32),
            pltpu.VMEM((_PPW,), jnp.int32),
            pltpu.VMEM((_PPW,), jnp.int32),
            pltpu.VMEM((_PPW + 16,), jnp.float32),
            pltpu.VMEM((_PPW + 16,), jnp.float32),
            pltpu.VMEM((_CHUNK, _D), jnp.float32),
            pltpu.VMEM((_CHUNK, _D), jnp.float32),
            pltpu.VMEM((_CHUNK, _D), jnp.float32),
            pltpu.SemaphoreType.DMA,
        ],
    )(ori, emb2d)
    return out.reshape(_N, 16, 64)
